# baseline (speedup 1.0000x reference)
import jax
import jax.numpy as jnp
from jax import lax
from jax.experimental import pallas as pl
from jax.experimental.pallas import tpu as pltpu
from jax.experimental.pallas import tpu_sc as plsc

N = 4096
F = 16
NC = 2
NS = 16
NW = NC * NS
S_ROWS = 1024
RPW = S_ROWS // NW
RPW_PAD = ((RPW + 15) // 16) * 16
ROW_BLOCK = 256
NEG = -1e30


def _accum_jg(jg, accs, cntv, adj_v, buf, pT_v, qs, fbase, nf):
    a = adj_v[buf, pl.ds(jg * 16, 16)]
    mask = a > 0.0
    z = jnp.where(mask, 0.0, NEG)
    if cntv is not None:
        cntv = cntv + jnp.where(mask, 1.0, 0.0)
    new = []
    for k in range(nf):
        f = fbase + k
        t = (pT_v[f, pl.ds(jg * 16, 16)] + z) - qs[f]
        new.append(accs[k] + jnp.maximum(t, 0.0))
    return tuple(new), cntv


def _row_compute(i, buf, adj_v, pT_v, q_v, out_v):
    qrow = q_v[i, :]
    qs = [qrow[f] for f in range(F)]
    FH = F // 2

    def body0(jg, carry):
        accs, cntv = carry[:FH], carry[FH]
        accs, cntv = _accum_jg(2 * jg, accs, cntv, adj_v, buf, pT_v, qs, 0, FH)
        accs, cntv = _accum_jg(2 * jg + 1, accs, cntv, adj_v, buf, pT_v, qs, 0, FH)
        return accs + (cntv,)

    def body1(jg, accs):
        accs, _ = _accum_jg(2 * jg, accs, None, adj_v, buf, pT_v, qs, FH, FH)
        accs, _ = _accum_jg(2 * jg + 1, accs, None, adj_v, buf, pT_v, qs, FH, FH)
        return accs

    zero = tuple(jnp.zeros((16,), jnp.float32) for _ in range(FH))
    res0 = lax.fori_loop(0, N // 32, body0, zero + (jnp.zeros((16,), jnp.float32),))
    res1 = lax.fori_loop(0, N // 32, body1, zero)
    accs, cntv = res0[:FH] + res1, res0[FH]

    lane = lax.iota(jnp.int32, 16)

    def lsum(v):
        for k in (8, 4, 2, 1):
            v = v + v[lane ^ k]
        return v

    total = jnp.zeros((16,), jnp.float32)
    for f in range(F):
        total = jnp.where(lane == f, lsum(accs[f]), total)
    cnts = lsum(cntv)
    inv = jnp.where(cnts > 0.0, 1.0 / jnp.maximum(cnts, 1.0), 0.0)
    out_v[i, :] = total * inv


def _sc_body(adj_hbm, cT_hbm, U_hbm, b_hbm, out_hbm,
             pT_v, q_v, adj_v, out_v, cT_v, U_v, b_v, sem0, sem1):
    wid = lax.axis_index("s") * NC + lax.axis_index("c")
    base = wid * RPW

    pltpu.sync_copy(cT_hbm, cT_v)
    pltpu.sync_copy(U_hbm, U_v)
    pltpu.sync_copy(b_hbm, b_v)

    u0 = U_v[0, :]
    u1 = U_v[1, :]
    bv = b_v[:]

    def ptj(jg, carry):
        c0 = cT_v[0, pl.ds(jg * 16, 16)]
        c1 = cT_v[1, pl.ds(jg * 16, 16)]
        for f in range(F):
            pT_v[f, pl.ds(jg * 16, 16)] = c0 * u0[f] + c1 * u1[f]
        return carry
    lax.fori_loop(0, N // 16, ptj, 0)

    lane16 = lax.iota(jnp.int32, 16)
    abase = pl.multiple_of((base // 16) * 16, 16)
    sh = base - abase

    def aligned16(row, off, g):
        return cT_v[row, pl.ds(pl.multiple_of(abase + off + g * 16, 16), 16)]

    def qrows(ig, carry):
        idxv = (lane16 + sh) % 16
        c0a = aligned16(0, 0, ig)[idxv]
        c0b = aligned16(0, 16, ig)[idxv]
        c1a = aligned16(1, 0, ig)[idxv]
        c1b = aligned16(1, 16, ig)[idxv]
        take_a = lane16 < (16 - sh)
        c0g = jnp.where(take_a, c0a, c0b)
        c1g = jnp.where(take_a, c1a, c1b)
        for l in range(16):
            q_v[ig * 16 + l, :] = c0g[l] * u0 + c1g[l] * u1 - bv
        return carry
    lax.fori_loop(0, RPW_PAD // 16, qrows, 0)

    pltpu.async_copy(adj_hbm.at[base], adj_v.at[0], sem0)

    def pair_body(k, carry):
        i0 = 2 * k
        pltpu.async_copy(adj_hbm.at[base + i0 + 1], adj_v.at[1], sem1)
        pltpu.make_async_copy(adj_hbm.at[base + i0], adj_v.at[0], sem0).wait()
        _row_compute(i0, 0, adj_v, pT_v, q_v, out_v)

        @pl.when(k < RPW // 2 - 1)
        def _prefetch():
            pltpu.async_copy(adj_hbm.at[base + i0 + 2], adj_v.at[0], sem0)

        pltpu.make_async_copy(adj_hbm.at[base + i0 + 1], adj_v.at[1], sem1).wait()
        _row_compute(i0 + 1, 1, adj_v, pT_v, q_v, out_v)
        return carry
    lax.fori_loop(0, RPW // 2, pair_body, 0)

    pltpu.sync_copy(out_v, out_hbm.at[wid])


def _sc_rows(adj, cT, U, b):
    mesh = plsc.VectorSubcoreMesh(
        core_axis_name="c", subcore_axis_name="s", num_cores=NC, num_subcores=NS)
    run = pl.kernel(
        _sc_body,
        out_type=jax.ShapeDtypeStruct((NW, RPW, F), jnp.float32),
        mesh=mesh,
        scratch_types=[
            pltpu.VMEM((F, N), jnp.float32),
            pltpu.VMEM((RPW_PAD, F), jnp.float32),
            pltpu.VMEM((2, N), jnp.float32),
            pltpu.VMEM((RPW, F), jnp.float32),
            pltpu.VMEM((2, N), jnp.float32),
            pltpu.VMEM((2, F), jnp.float32),
            pltpu.VMEM((F,), jnp.float32),
            pltpu.SemaphoreType.DMA,
            pltpu.SemaphoreType.DMA,
        ],
    )
    return run(adj, cT, U, b.reshape(F)).reshape(S_ROWS, F)


def _tc_body(adj_ref, crows_ref, cT_ref, UT_ref, U_ref, b_ref, out_ref):
    m = (adj_ref[:] > 0).astype(jnp.float32)
    cnt = jnp.sum(m, axis=1, keepdims=True)
    pT = jnp.dot(UT_ref[:], cT_ref[:], preferred_element_type=jnp.float32)
    q = jnp.dot(crows_ref[:], U_ref[:], preferred_element_type=jnp.float32) - b_ref[:]
    cols = []
    for f in range(F):
        pj = pT[f, :][None, :]
        qf = q[:, f][:, None]
        contrib = jnp.maximum(pj - qf, 0.0) * m
        cols.append(jnp.sum(contrib, axis=1, keepdims=True))
    acc = jnp.concatenate(cols, axis=1)
    mean = acc / jnp.maximum(cnt, 1.0)
    out_ref[:] = jnp.where(cnt > 0, mean, 0.0)


def _tc_rows(adj, coords, cT, U, b):
    nrows = N - S_ROWS
    off = S_ROWS // ROW_BLOCK
    grid = (nrows // ROW_BLOCK,)
    return pl.pallas_call(
        _tc_body,
        grid=grid,
        in_specs=[
            pl.BlockSpec((ROW_BLOCK, N), lambda i: (i + off, 0)),
            pl.BlockSpec((ROW_BLOCK, 2), lambda i: (i + off, 0)),
            pl.BlockSpec((2, N), lambda i: (0, 0)),
            pl.BlockSpec((F, 2), lambda i: (0, 0)),
            pl.BlockSpec((2, F), lambda i: (0, 0)),
            pl.BlockSpec((1, F), lambda i: (0, 0)),
        ],
        out_specs=pl.BlockSpec((ROW_BLOCK, F), lambda i: (i, 0)),
        out_shape=jax.ShapeDtypeStruct((nrows, F), jnp.float32),
        compiler_params=pltpu.CompilerParams(
            dimension_semantics=("parallel",),
        ),
    )(adj, coords, cT, U.T, U, b.reshape(1, F))


@jax.jit
def kernel(x, adj, coords, U, b):
    del x
    coords = coords.astype(jnp.float32)
    cT = coords.T
    out_sc = _sc_rows(adj, cT, U, b)
    out_tc = _tc_rows(adj, coords, cT, U, b)
    return jnp.concatenate([out_sc, out_tc], axis=0)


# speedup vs baseline: 1.0193x; 1.0193x over previous
import jax
import jax.numpy as jnp
from jax import lax
from jax.experimental import pallas as pl
from jax.experimental.pallas import tpu as pltpu
from jax.experimental.pallas import tpu_sc as plsc

N = 4096
F = 16
NC = 2
NS = 16
NW = NC * NS
S_ROWS = 768
RPW = S_ROWS // NW
RPW_PAD = ((RPW + 15) // 16) * 16
ROW_BLOCK = 256
NEG = -1e30


def _accum_jg(jg, accs, cntv, adj_v, buf, pT_v, qs):
    a = adj_v[buf, pl.ds(jg * 16, 16)]
    mask = a > 0.0
    z = jnp.where(mask, 0.0, NEG)
    cntv = cntv + jnp.where(mask, 1.0, 0.0)
    new = []
    for f in range(F):
        t = (pT_v[f, pl.ds(jg * 16, 16)] + z) - qs[f]
        new.append(accs[f] + jnp.maximum(t, 0.0))
    return tuple(new), cntv


def _row_compute(i, buf, adj_v, pT_v, q_v, out_v):
    qrow = q_v[i, :]
    qs = [qrow[f] for f in range(F)]

    def jg_body(jg, carry):
        accs, cntv = carry[:F], carry[F]
        accs, cntv = _accum_jg(2 * jg, accs, cntv, adj_v, buf, pT_v, qs)
        accs, cntv = _accum_jg(2 * jg + 1, accs, cntv, adj_v, buf, pT_v, qs)
        return accs + (cntv,)

    init = tuple(jnp.zeros((16,), jnp.float32) for _ in range(F + 1))
    res = lax.fori_loop(0, N // 32, jg_body, init)
    accs, cntv = res[:F], res[F]

    lane = lax.iota(jnp.int32, 16)

    def lsum(v):
        for k in (8, 4, 2, 1):
            v = v + v[lane ^ k]
        return v

    total = jnp.zeros((16,), jnp.float32)
    for f in range(F):
        total = jnp.where(lane == f, lsum(accs[f]), total)
    cnts = lsum(cntv)
    inv = jnp.where(cnts > 0.0, 1.0 / jnp.maximum(cnts, 1.0), 0.0)
    out_v[i, :] = total * inv


def _sc_body(adj_hbm, cT_hbm, U_hbm, b_hbm, out_hbm,
             pT_v, q_v, adj_v, out_v, cT_v, U_v, b_v, sem0, sem1):
    wid = lax.axis_index("s") * NC + lax.axis_index("c")
    base = wid * RPW

    pltpu.sync_copy(cT_hbm, cT_v)
    pltpu.sync_copy(U_hbm, U_v)
    pltpu.sync_copy(b_hbm, b_v)

    u0 = U_v[0, :]
    u1 = U_v[1, :]
    bv = b_v[:]

    def ptj(jg, carry):
        c0 = cT_v[0, pl.ds(jg * 16, 16)]
        c1 = cT_v[1, pl.ds(jg * 16, 16)]
        for f in range(F):
            pT_v[f, pl.ds(jg * 16, 16)] = c0 * u0[f] + c1 * u1[f]
        return carry
    lax.fori_loop(0, N // 16, ptj, 0)

    lane16 = lax.iota(jnp.int32, 16)
    abase = pl.multiple_of((base // 16) * 16, 16)
    sh = base - abase

    def aligned16(row, off, g):
        return cT_v[row, pl.ds(pl.multiple_of(abase + off + g * 16, 16), 16)]

    def qrows(ig, carry):
        idxv = (lane16 + sh) % 16
        c0a = aligned16(0, 0, ig)[idxv]
        c0b = aligned16(0, 16, ig)[idxv]
        c1a = aligned16(1, 0, ig)[idxv]
        c1b = aligned16(1, 16, ig)[idxv]
        take_a = lane16 < (16 - sh)
        c0g = jnp.where(take_a, c0a, c0b)
        c1g = jnp.where(take_a, c1a, c1b)
        for l in range(16):
            q_v[ig * 16 + l, :] = c0g[l] * u0 + c1g[l] * u1 - bv
        return carry
    lax.fori_loop(0, RPW_PAD // 16, qrows, 0)

    pltpu.async_copy(adj_hbm.at[base], adj_v.at[0], sem0)

    def pair_body(k, carry):
        i0 = 2 * k
        pltpu.async_copy(adj_hbm.at[base + i0 + 1], adj_v.at[1], sem1)
        pltpu.make_async_copy(adj_hbm.at[base + i0], adj_v.at[0], sem0).wait()
        _row_compute(i0, 0, adj_v, pT_v, q_v, out_v)

        @pl.when(k < RPW // 2 - 1)
        def _prefetch():
            pltpu.async_copy(adj_hbm.at[base + i0 + 2], adj_v.at[0], sem0)

        pltpu.make_async_copy(adj_hbm.at[base + i0 + 1], adj_v.at[1], sem1).wait()
        _row_compute(i0 + 1, 1, adj_v, pT_v, q_v, out_v)
        return carry
    lax.fori_loop(0, RPW // 2, pair_body, 0)

    pltpu.sync_copy(out_v, out_hbm.at[wid])


def _sc_rows(adj, cT, U, b):
    mesh = plsc.VectorSubcoreMesh(
        core_axis_name="c", subcore_axis_name="s", num_cores=NC, num_subcores=NS)
    run = pl.kernel(
        _sc_body,
        out_type=jax.ShapeDtypeStruct((NW, RPW, F), jnp.float32),
        mesh=mesh,
        scratch_types=[
            pltpu.VMEM((F, N), jnp.float32),
            pltpu.VMEM((RPW_PAD, F), jnp.float32),
            pltpu.VMEM((2, N), jnp.float32),
            pltpu.VMEM((RPW, F), jnp.float32),
            pltpu.VMEM((2, N), jnp.float32),
            pltpu.VMEM((2, F), jnp.float32),
            pltpu.VMEM((F,), jnp.float32),
            pltpu.SemaphoreType.DMA,
            pltpu.SemaphoreType.DMA,
        ],
    )
    return run(adj, cT, U, b.reshape(F)).reshape(S_ROWS, F)


def _tc_body(adj_ref, crows_ref, cT_ref, UT_ref, U_ref, b_ref, out_ref):
    m = (adj_ref[:] > 0).astype(jnp.float32)
    cnt = jnp.sum(m, axis=1, keepdims=True)
    pT = jnp.dot(UT_ref[:], cT_ref[:], preferred_element_type=jnp.float32)
    q = jnp.dot(crows_ref[:], U_ref[:], preferred_element_type=jnp.float32) - b_ref[:]
    cols = []
    for f in range(F):
        pj = pT[f, :][None, :]
        qf = q[:, f][:, None]
        contrib = jnp.maximum(pj - qf, 0.0) * m
        cols.append(jnp.sum(contrib, axis=1, keepdims=True))
    acc = jnp.concatenate(cols, axis=1)
    mean = acc / jnp.maximum(cnt, 1.0)
    out_ref[:] = jnp.where(cnt > 0, mean, 0.0)


def _tc_rows(adj, coords, cT, U, b):
    nrows = N - S_ROWS
    off = S_ROWS // ROW_BLOCK
    grid = (nrows // ROW_BLOCK,)
    return pl.pallas_call(
        _tc_body,
        grid=grid,
        in_specs=[
            pl.BlockSpec((ROW_BLOCK, N), lambda i: (i + off, 0)),
            pl.BlockSpec((ROW_BLOCK, 2), lambda i: (i + off, 0)),
            pl.BlockSpec((2, N), lambda i: (0, 0)),
            pl.BlockSpec((F, 2), lambda i: (0, 0)),
            pl.BlockSpec((2, F), lambda i: (0, 0)),
            pl.BlockSpec((1, F), lambda i: (0, 0)),
        ],
        out_specs=pl.BlockSpec((ROW_BLOCK, F), lambda i: (i, 0)),
        out_shape=jax.ShapeDtypeStruct((nrows, F), jnp.float32),
        compiler_params=pltpu.CompilerParams(
            dimension_semantics=("parallel",),
        ),
    )(adj, coords, cT, U.T, U, b.reshape(1, F))


@jax.jit
def kernel(x, adj, coords, U, b):
    del x
    coords = coords.astype(jnp.float32)
    cT = coords.T
    out_sc = _sc_rows(adj, cT, U, b)
    out_tc = _tc_rows(adj, coords, cT, U, b)
    return jnp.concatenate([out_sc, out_tc], axis=0)
